# baseline (speedup 1.0000x reference)
import functools

import jax
import jax.numpy as jnp
from jax import lax
from jax.experimental import pallas as pl
from jax.experimental.pallas import tpu as pltpu
from jax.experimental.pallas import tpu_sc as plsc

N_VOCAB = 100000
N_EMBED = 64
BATCH = 16384

NUM_CORES = 2
NUM_SUBCORES = 16
NUM_WORKERS = NUM_CORES * NUM_SUBCORES
CHUNK = 128
CHUNKS_PER_WORKER = BATCH // (NUM_WORKERS * CHUNK)

_mesh = plsc.VectorSubcoreMesh(core_axis_name="c", subcore_axis_name="s")


@functools.partial(
    pl.kernel,
    mesh=_mesh,
    out_type=jax.ShapeDtypeStruct(
        (NUM_WORKERS * CHUNKS_PER_WORKER, CHUNK, N_EMBED), jnp.float32
    ),
    scratch_types=[
        pltpu.VMEM((CHUNKS_PER_WORKER, CHUNK), jnp.int32),
        pltpu.VMEM((CHUNKS_PER_WORKER, CHUNK, N_EMBED), jnp.float32),
        pltpu.SemaphoreType.DMA,
    ],
    compiler_params=pltpu.CompilerParams(use_tc_tiling_on_sc=False),
)
def _embed_gather(idx_hbm, table_hbm, out_hbm, idx_v, rows_v, sem):
    wid = lax.axis_index("s") * NUM_CORES + lax.axis_index("c")
    base = wid * CHUNKS_PER_WORKER
    pltpu.sync_copy(idx_hbm.at[pl.ds(base, CHUNKS_PER_WORKER)], idx_v)
    copies = []
    for j in range(CHUNKS_PER_WORKER):
        copies.append(
            pltpu.async_copy(table_hbm.at[idx_v.at[j]], rows_v.at[j], sem)
        )
    for c in copies:
        c.wait()
    pltpu.sync_copy(rows_v, out_hbm.at[pl.ds(base, CHUNKS_PER_WORKER)])


def kernel(input_words, W_in):
    idx = input_words.reshape(NUM_WORKERS * CHUNKS_PER_WORKER, CHUNK)
    out = _embed_gather(idx, W_in)
    return out.reshape(BATCH, N_EMBED)


# speedup vs baseline: 1.4975x; 1.4975x over previous
import functools

import jax
import jax.numpy as jnp
from jax import lax
from jax.experimental import pallas as pl
from jax.experimental.pallas import tpu as pltpu
from jax.experimental.pallas import tpu_sc as plsc

N_VOCAB = 100000
N_EMBED = 64
BATCH = 16384

NUM_CORES = 2
NUM_SUBCORES = 16
NUM_WORKERS = NUM_CORES * NUM_SUBCORES
ROWS_PER_WORKER = BATCH // NUM_WORKERS

_mesh = plsc.VectorSubcoreMesh(core_axis_name="c", subcore_axis_name="s")


@functools.partial(
    pl.kernel,
    mesh=_mesh,
    out_type=jax.ShapeDtypeStruct((BATCH, N_EMBED), jnp.float32),
    scratch_types=[
        pltpu.VMEM((ROWS_PER_WORKER,), jnp.int32),
        pltpu.VMEM((ROWS_PER_WORKER, N_EMBED), jnp.float32),
        pltpu.SemaphoreType.DMA,
        pltpu.SemaphoreType.DMA,
    ],
)
def _embed_gather(idx_hbm, table_hbm, out_hbm, idx_v, rows_v, sem_i, sem):
    wid = lax.axis_index("s") * NUM_CORES + lax.axis_index("c")
    base = wid * ROWS_PER_WORKER
    pltpu.async_copy(idx_hbm.at[pl.ds(base, ROWS_PER_WORKER)], idx_v, sem_i).wait()

    def body(c, carry):
        v = idx_v[pl.ds(c * 16, 16)]
        for lane in range(16):
            r = v[lane]
            i = c * 16 + lane
            pltpu.async_copy(
                table_hbm.at[pl.ds(r, 1)], rows_v.at[pl.ds(i, 1)], sem
            )
        return carry

    lax.fori_loop(0, ROWS_PER_WORKER // 16, body, 0)
    pltpu.make_async_copy(table_hbm.at[pl.ds(0, ROWS_PER_WORKER)], rows_v, sem).wait()
    pltpu.sync_copy(rows_v, out_hbm.at[pl.ds(base, ROWS_PER_WORKER)])


def kernel(input_words, W_in):
    return _embed_gather(input_words, W_in)


# speedup vs baseline: 1.9842x; 1.3250x over previous
import functools

import jax
import jax.numpy as jnp
from jax import lax
from jax.experimental import pallas as pl
from jax.experimental.pallas import tpu as pltpu
from jax.experimental.pallas import tpu_sc as plsc

N_VOCAB = 100000
N_EMBED = 64
BATCH = 16384

NUM_CORES = 2
NUM_SUBCORES = 16
NUM_WORKERS = NUM_CORES * NUM_SUBCORES
ROWS_PER_WORKER = N_EMBED // NUM_WORKERS
OUT_CHUNK = 8192

_mesh = plsc.VectorSubcoreMesh(core_axis_name="c", subcore_axis_name="s")


@functools.partial(
    pl.kernel,
    mesh=_mesh,
    out_type=jax.ShapeDtypeStruct((N_EMBED, BATCH), jnp.float32),
    scratch_types=[
        pltpu.VMEM((BATCH,), jnp.int32),
        pltpu.VMEM((1, N_VOCAB), jnp.float32),
        pltpu.VMEM((1, OUT_CHUNK), jnp.float32),
        pltpu.SemaphoreType.DMA,
        pltpu.SemaphoreType.DMA,
    ],
    compiler_params=pltpu.CompilerParams(needs_layout_passes=False),
)
def _embed_gather_t(idx_hbm, table_t_hbm, out_t_hbm, idx_v, row_v, out_v, sem_i, sem):
    wid = lax.axis_index("s") * NUM_CORES + lax.axis_index("c")
    pltpu.async_copy(idx_hbm, idx_v, sem_i).wait()

    zeros16 = jnp.zeros((16,), jnp.int32)

    for jr in range(ROWS_PER_WORKER):
        j = wid + jr * NUM_WORKERS
        pltpu.async_copy(table_t_hbm.at[pl.ds(j, 1)], row_v, sem).wait()
        for h in range(BATCH // OUT_CHUNK):
            def body(c, carry):
                iv = idx_v[pl.ds(h * OUT_CHUNK + c * 16, 16)]
                vals = plsc.load_gather(row_v, [zeros16, iv])
                out_v[0, pl.ds(c * 16, 16)] = vals
                return carry

            lax.fori_loop(0, OUT_CHUNK // 16, body, 0)
            pltpu.sync_copy(
                out_v, out_t_hbm.at[pl.ds(j, 1), pl.ds(h * OUT_CHUNK, OUT_CHUNK)]
            )


def kernel(input_words, W_in):
    out_t = _embed_gather_t(input_words, W_in.T)
    return out_t.T


# speedup vs baseline: 2.6758x; 1.3486x over previous
import functools

import jax
import jax.numpy as jnp
from jax import lax
from jax.experimental import pallas as pl
from jax.experimental.pallas import tpu as pltpu
from jax.experimental.pallas import tpu_sc as plsc

N_VOCAB = 100000
N_EMBED = 64
BATCH = 16384

NUM_CORES = 2
NUM_SUBCORES = 16
NUM_WORKERS = NUM_CORES * NUM_SUBCORES
ROWS_PER_WORKER = N_EMBED // NUM_WORKERS
OUT_CHUNK = 8192

_mesh = plsc.VectorSubcoreMesh(core_axis_name="c", subcore_axis_name="s")


@functools.partial(
    pl.kernel,
    mesh=_mesh,
    out_type=jax.ShapeDtypeStruct((N_EMBED, BATCH), jnp.float32),
    scratch_types=[
        pltpu.VMEM((BATCH,), jnp.int32),
        pltpu.VMEM((1, N_VOCAB), jnp.float32),
        pltpu.VMEM((1, OUT_CHUNK), jnp.float32),
        pltpu.SemaphoreType.DMA,
        pltpu.SemaphoreType.DMA,
    ],
    compiler_params=pltpu.CompilerParams(needs_layout_passes=False),
)
def _embed_gather_t(idx_hbm, table_t_hbm, out_t_hbm, idx_v, row_v, out_v, sem_i, sem):
    wid = lax.axis_index("s") * NUM_CORES + lax.axis_index("c")
    pltpu.async_copy(idx_hbm, idx_v, sem_i).wait()

    zeros16 = jnp.zeros((16,), jnp.int32)

    for jr in range(ROWS_PER_WORKER):
        j = wid + jr * NUM_WORKERS
        pltpu.async_copy(table_t_hbm.at[pl.ds(j, 1)], row_v, sem).wait()
        for h in range(BATCH // OUT_CHUNK):
            @plsc.parallel_loop(0, OUT_CHUNK // 16, unroll=8)
            def body(c):
                iv = idx_v[pl.ds(h * OUT_CHUNK + c * 16, 16)]
                vals = plsc.load_gather(row_v, [zeros16, iv])
                out_v[0, pl.ds(c * 16, 16)] = vals
            pltpu.sync_copy(
                out_v, out_t_hbm.at[pl.ds(j, 1), pl.ds(h * OUT_CHUNK, OUT_CHUNK)]
            )


def kernel(input_words, W_in):
    out_t = _embed_gather_t(input_words, W_in.T)
    return out_t.T


# speedup vs baseline: 2.6862x; 1.0039x over previous
import functools

import jax
import jax.numpy as jnp
from jax import lax
from jax.experimental import pallas as pl
from jax.experimental.pallas import tpu as pltpu
from jax.experimental.pallas import tpu_sc as plsc

N_VOCAB = 100000
N_EMBED = 64
BATCH = 16384

NUM_CORES = 2
NUM_SUBCORES = 16
NUM_WORKERS = NUM_CORES * NUM_SUBCORES
ROWS_PER_WORKER = N_EMBED // NUM_WORKERS
OUT_CHUNK = 8192

_mesh = plsc.VectorSubcoreMesh(core_axis_name="c", subcore_axis_name="s")


@functools.partial(
    pl.kernel,
    mesh=_mesh,
    out_type=jax.ShapeDtypeStruct((N_EMBED, BATCH), jnp.float32),
    scratch_types=[
        pltpu.VMEM((BATCH,), jnp.int32),
        pltpu.VMEM((1, N_VOCAB), jnp.float32),
        pltpu.VMEM((1, OUT_CHUNK), jnp.float32),
        pltpu.SemaphoreType.DMA,
        pltpu.SemaphoreType.DMA,
    ],
    compiler_params=pltpu.CompilerParams(
        needs_layout_passes=False, skip_device_barrier=True
    ),
)
def _embed_gather_t(idx_hbm, table_t_hbm, out_t_hbm, idx_v, row_v, out_v, sem_i, sem):
    wid = lax.axis_index("s") * NUM_CORES + lax.axis_index("c")
    pltpu.async_copy(idx_hbm, idx_v, sem_i).wait()

    zeros16 = jnp.zeros((16,), jnp.int32)

    for jr in range(ROWS_PER_WORKER):
        j = wid + jr * NUM_WORKERS
        pltpu.async_copy(table_t_hbm.at[pl.ds(j, 1)], row_v, sem).wait()
        for h in range(BATCH // OUT_CHUNK):
            @plsc.parallel_loop(0, OUT_CHUNK // 16, unroll=8)
            def body(c):
                iv = idx_v[pl.ds(h * OUT_CHUNK + c * 16, 16)]
                vals = plsc.load_gather(row_v, [zeros16, iv])
                out_v[0, pl.ds(c * 16, 16)] = vals
            pltpu.sync_copy(
                out_v, out_t_hbm.at[pl.ds(j, 1), pl.ds(h * OUT_CHUNK, OUT_CHUNK)]
            )


def kernel(input_words, W_in):
    out_t = _embed_gather_t(input_words, W_in.T)
    return out_t.T


# speedup vs baseline: 2.7017x; 1.0057x over previous
import functools

import jax
import jax.numpy as jnp
from jax import lax
from jax.experimental import pallas as pl
from jax.experimental.pallas import tpu as pltpu
from jax.experimental.pallas import tpu_sc as plsc

N_VOCAB = 100000
N_EMBED = 64
BATCH = 16384

NUM_CORES = 2
NUM_SUBCORES = 16
NUM_WORKERS = NUM_CORES * NUM_SUBCORES
ROWS_PER_WORKER = N_EMBED // NUM_WORKERS
OUT_CHUNK = 8192

_mesh = plsc.VectorSubcoreMesh(core_axis_name="c", subcore_axis_name="s")


@functools.partial(
    pl.kernel,
    mesh=_mesh,
    out_type=jax.ShapeDtypeStruct((N_EMBED, BATCH), jnp.float32),
    scratch_types=[
        pltpu.VMEM((BATCH,), jnp.int32),
        pltpu.VMEM((1, N_VOCAB), jnp.float32),
        pltpu.VMEM((1, OUT_CHUNK), jnp.float32),
        pltpu.SemaphoreType.DMA,
        pltpu.SemaphoreType.DMA,
    ],
    compiler_params=pltpu.CompilerParams(needs_layout_passes=False),
)
def _embed_gather_t(idx_hbm, table_t_hbm, out_t_hbm, idx_v, row_v, out_v, sem_i, sem):
    wid = lax.axis_index("s") * NUM_CORES + lax.axis_index("c")
    pltpu.async_copy(idx_hbm, idx_v, sem_i).wait()

    zeros16 = jnp.zeros((16,), jnp.int32)

    def row_body(jr, carry):
        j = wid + jr * NUM_WORKERS
        pltpu.async_copy(table_t_hbm.at[pl.ds(j, 1)], row_v, sem).wait()

        def chunk_body(h, c2):
            @plsc.parallel_loop(0, OUT_CHUNK // 16, unroll=8)
            def body(c):
                iv = idx_v[pl.ds(h * OUT_CHUNK + c * 16, 16)]
                vals = plsc.load_gather(row_v, [zeros16, iv])
                out_v[0, pl.ds(c * 16, 16)] = vals

            pltpu.sync_copy(
                out_v, out_t_hbm.at[pl.ds(j, 1), pl.ds(h * OUT_CHUNK, OUT_CHUNK)]
            )
            return c2

        lax.fori_loop(0, BATCH // OUT_CHUNK, chunk_body, 0)
        return carry

    lax.fori_loop(0, ROWS_PER_WORKER, row_body, 0)


def kernel(input_words, W_in):
    out_t = _embed_gather_t(input_words, W_in.T)
    return out_t.T
